# baseline (speedup 1.0000x reference)
import dataclasses
import functools

import jax
import jax.numpy as jnp
from jax import lax
from jax.experimental import pallas as pl
from jax.experimental.pallas import tpu as pltpu
from jax.experimental.pallas import tpu_sc as plsc

NC = 2
NS = 16
NW = NC * NS
LANES = 16
CHUNK = 128
NSLOT = 2
NACC = 10240


def _sc_aggregate(xb, src, dst, zrow, z16, iota):
    d = xb.shape[1]
    nchunk = src.shape[1]
    ngroup = nchunk // NSLOT
    nrow16 = NACC // LANES

    mesh = plsc.VectorSubcoreMesh(core_axis_name="c", subcore_axis_name="s")

    cp = pltpu.CompilerParams()
    if "needs_layout_passes" in pltpu.CompilerParams.__dataclass_fields__:
        cp = dataclasses.replace(cp, needs_layout_passes=False)
    if "use_tc_tiling_on_sc" in pltpu.CompilerParams.__dataclass_fields__:
        cp = dataclasses.replace(cp, use_tc_tiling_on_sc=False)

    @functools.partial(
        pl.kernel,
        compiler_params=cp,
        out_type=[
            jax.ShapeDtypeStruct((NC, NACC, d), jnp.bfloat16),
            jax.ShapeDtypeStruct((NC, nrow16, LANES), jnp.float32),
        ],
        mesh=mesh,
        scratch_types=[
            pltpu.VMEM((nchunk, CHUNK), jnp.int32),
            pltpu.VMEM((nchunk, CHUNK), jnp.int32),
            pltpu.VMEM((NSLOT, CHUNK, 128), jnp.bfloat16),
            pltpu.VMEM((nrow16, LANES), jnp.float32),
            pltpu.VMEM((CHUNK,), jnp.int32),
            pltpu.VMEM_SHARED((NACC, 128), jnp.bfloat16),
            pltpu.VMEM_SHARED((nrow16, LANES), jnp.float32),
            [pltpu.SemaphoreType.DMA] * NSLOT,
            [pltpu.SemaphoreType.DMA] * NSLOT,
        ],
    )
    def sc_kernel(x_hbm, src_hbm, dst_hbm, zrow_hbm, z16_hbm, iota_hbm,
                  acc_out, cnt_out, sidx_all, didx_all, rows_v, cnt_v,
                  idxc_v, acc_sh, cnt_sh, sem_g, sem_s):
        cid = lax.axis_index("c")
        sid = lax.axis_index("s")
        wid = cid * NS + sid
        rpt = NACC // NS

        pltpu.sync_copy(zrow_hbm, acc_sh.at[pl.ds(sid * rpt, rpt)])
        pltpu.sync_copy(z16_hbm, cnt_v)
        pltpu.sync_copy(src_hbm.at[wid], sidx_all)
        pltpu.sync_copy(dst_hbm.at[wid], didx_all)

        @pl.when(sid == 0)
        def _():
            pltpu.sync_copy(z16_hbm, cnt_sh)

        plsc.subcore_barrier()

        ones = jnp.full((LANES,), 1.0, jnp.float32)
        four = jnp.full((LANES,), 4, jnp.int32)
        fifteen = jnp.full((LANES,), 15, jnp.int32)

        def gather_start(c, b):
            pltpu.async_copy(x_hbm.at[sidx_all.at[c]], rows_v.at[b], sem_g[b])

        def gather_wait(c, b):
            pltpu.make_async_copy(x_hbm.at[sidx_all.at[c]], rows_v.at[b],
                                  sem_g[b]).wait()

        def scatter_start(c, b):
            pltpu.async_copy(rows_v.at[b], acc_sh.at[didx_all.at[c]],
                             sem_s[b], add=True)

        def scatter_wait(c, b):
            pltpu.make_async_copy(rows_v.at[b], acc_sh.at[didx_all.at[c]],
                                  sem_s[b]).wait()

        def counts(c):
            for i in range(CHUNK // LANES):
                dv = didx_all[c, pl.ds(i * LANES, LANES)]
                row = lax.shift_right_logical(dv, four)
                col = lax.bitwise_and(dv, fifteen)
                plsc.addupdate_scatter(cnt_v, [row, col], ones)

        gather_start(0, 0)

        @pl.loop(0, ngroup)
        def _(g):
            c0 = g * 2
            c1 = c0 + 1

            @pl.when(g > 0)
            def _():
                scatter_wait(c0 - 1, 1)

            gather_start(c1, 1)
            gather_wait(c0, 0)
            scatter_start(c0, 0)
            counts(c0)
            scatter_wait(c0, 0)

            @pl.when(g + 1 < ngroup)
            def _():
                gather_start(c0 + 2, 0)

            gather_wait(c1, 1)
            scatter_start(c1, 1)
            counts(c1)

        scatter_wait(nchunk - 1, 1)

        plsc.subcore_barrier()

        for c in range(nrow16 // CHUNK):
            pltpu.sync_copy(iota_hbm.at[pl.ds(c * CHUNK, CHUNK)], idxc_v)
            pltpu.sync_copy(cnt_v.at[pl.ds(c * CHUNK, CHUNK)],
                            cnt_sh.at[idxc_v], add=True)

        pltpu.sync_copy(acc_sh.at[pl.ds(sid * rpt, rpt)],
                        acc_out.at[cid, pl.ds(sid * rpt, rpt)])

        plsc.subcore_barrier()

        crows = nrow16 // NS
        pltpu.sync_copy(cnt_sh.at[pl.ds(sid * crows, crows)],
                        cnt_out.at[cid, pl.ds(sid * crows, crows)])

    return sc_kernel(xb, src, dst, zrow, z16, iota)


def _tc_dense(p, cnt, x, wl, wr, b):
    n, d = x.shape
    blk = 2000

    def body(p_ref, c_ref, x_ref, wl_ref, wr_ref, b_ref, o_ref):
        c = jnp.clip(c_ref[0] + c_ref[1], 1.0)
        s = p_ref[0].astype(jnp.float32) + p_ref[1].astype(jnp.float32)
        m = s / c
        o_ref[...] = (
            jnp.dot(m, wl_ref[...], preferred_element_type=jnp.float32,
                    precision=lax.Precision.HIGHEST)
            + jnp.dot(x_ref[...], wr_ref[...], preferred_element_type=jnp.float32,
                      precision=lax.Precision.HIGHEST)
            + b_ref[...])

    return pl.pallas_call(
        body,
        grid=(n // blk,),
        in_specs=[
            pl.BlockSpec((NC, blk, d), lambda i: (0, i, 0)),
            pl.BlockSpec((NC, blk, 1), lambda i: (0, i, 0)),
            pl.BlockSpec((blk, d), lambda i: (i, 0)),
            pl.BlockSpec((d, d), lambda i: (0, 0)),
            pl.BlockSpec((d, d), lambda i: (0, 0)),
            pl.BlockSpec((1, d), lambda i: (0, 0)),
        ],
        out_specs=pl.BlockSpec((blk, d), lambda i: (i, 0)),
        out_shape=jax.ShapeDtypeStruct((n, d), jnp.float32),
    )(p, cnt, x, wl, wr, b.reshape(1, d))


def kernel(x, edge_index, W_l, W_r, b):
    n, d = x.shape
    e = edge_index.shape[1]
    nchunk = -(-e // (NW * NSLOT * CHUNK)) * NSLOT
    ept = nchunk * CHUNK
    epad = ept * NW
    src = edge_index[0]
    dst = edge_index[1]
    if epad > e:
        pad = epad - e
        src = jnp.concatenate([src, jnp.zeros((pad,), jnp.int32)])
        dst = jnp.concatenate([dst, jnp.full((pad,), NACC - 1, jnp.int32)])
    src = src.reshape(NW, nchunk, CHUNK)
    dst = dst.reshape(NW, nchunk, CHUNK)

    xb = x.astype(jnp.bfloat16)
    zrow = jnp.zeros((NACC // NS, d), jnp.bfloat16)
    z16 = jnp.zeros((NACC // LANES, LANES), jnp.float32)
    iota = jnp.arange(NACC // LANES, dtype=jnp.int32)

    acc, cnt = _sc_aggregate(xb, src, dst, zrow, z16, iota)
    cnt = cnt.reshape(NC, NACC, 1)
    return _tc_dense(acc, cnt, x, W_l, W_r, b)


# speedup vs baseline: 1.5454x; 1.5454x over previous
import dataclasses
import functools

import jax
import jax.numpy as jnp
from jax import lax
from jax.experimental import pallas as pl
from jax.experimental.pallas import tpu as pltpu
from jax.experimental.pallas import tpu_sc as plsc

NC = 2
NS = 16
LANES = 16
CHUNK = 128
NSLOT = 2
NACC = 10240
DH = 64


def _sc_aggregate(xs, src, dst, z64, z16, iota):
    nchunk = src.shape[1]
    ngroup = nchunk // NSLOT
    nrow16 = NACC // LANES

    mesh = plsc.VectorSubcoreMesh(core_axis_name="c", subcore_axis_name="s")

    cp = pltpu.CompilerParams()
    if "needs_layout_passes" in pltpu.CompilerParams.__dataclass_fields__:
        cp = dataclasses.replace(cp, needs_layout_passes=False)
    if "use_tc_tiling_on_sc" in pltpu.CompilerParams.__dataclass_fields__:
        cp = dataclasses.replace(cp, use_tc_tiling_on_sc=False)

    @functools.partial(
        pl.kernel,
        compiler_params=cp,
        out_type=[
            jax.ShapeDtypeStruct((NC, NACC, DH), jnp.bfloat16),
            jax.ShapeDtypeStruct((NC, nrow16, LANES), jnp.float32),
        ],
        mesh=mesh,
        scratch_types=[
            pltpu.VMEM((nchunk, CHUNK), jnp.int32),
            pltpu.VMEM((nchunk, CHUNK), jnp.int32),
            pltpu.VMEM((NSLOT, CHUNK, DH), jnp.bfloat16),
            pltpu.VMEM((nrow16, LANES), jnp.float32),
            pltpu.VMEM((CHUNK,), jnp.int32),
            pltpu.VMEM_SHARED((NACC, DH), jnp.bfloat16),
            pltpu.VMEM_SHARED((nrow16, LANES), jnp.float32),
            [pltpu.SemaphoreType.DMA] * NSLOT,
            [pltpu.SemaphoreType.DMA] * NSLOT,
        ],
    )
    def sc_kernel(xs_hbm, src_hbm, dst_hbm, z64_hbm, z16_hbm, iota_hbm,
                  acc_out, cnt_out, sidx_all, didx_all, rows_v, cnt_v,
                  idxc_v, acc_sh, cnt_sh, sem_g, sem_s):
        cid = lax.axis_index("c")
        sid = lax.axis_index("s")
        rpt = NACC // NS
        xh = xs_hbm.at[cid]

        pltpu.sync_copy(z64_hbm, acc_sh.at[pl.ds(sid * rpt, rpt)])
        pltpu.sync_copy(z16_hbm, cnt_v)
        pltpu.sync_copy(src_hbm.at[sid], sidx_all)
        pltpu.sync_copy(dst_hbm.at[sid], didx_all)

        @pl.when(sid == 0)
        def _():
            pltpu.sync_copy(z16_hbm, cnt_sh)

        plsc.subcore_barrier()

        ones = jnp.full((LANES,), 1.0, jnp.float32)
        four = jnp.full((LANES,), 4, jnp.int32)
        fifteen = jnp.full((LANES,), 15, jnp.int32)

        def gather_start(c, b):
            pltpu.async_copy(xh.at[sidx_all.at[c]], rows_v.at[b], sem_g[b])

        def gather_wait(c, b):
            pltpu.make_async_copy(xh.at[sidx_all.at[c]], rows_v.at[b],
                                  sem_g[b]).wait()

        def scatter_start(c, b):
            pltpu.async_copy(rows_v.at[b], acc_sh.at[didx_all.at[c]],
                             sem_s[b], add=True)

        def scatter_wait(c, b):
            pltpu.make_async_copy(rows_v.at[b], acc_sh.at[didx_all.at[c]],
                                  sem_s[b]).wait()

        def counts(c):
            @pl.when(lax.bitwise_and(c, 1) == cid)
            def _():
                for i in range(CHUNK // LANES):
                    dv = didx_all[c, pl.ds(i * LANES, LANES)]
                    row = lax.shift_right_logical(dv, four)
                    col = lax.bitwise_and(dv, fifteen)
                    plsc.addupdate_scatter(cnt_v, [row, col], ones)

        gather_start(0, 0)

        @pl.loop(0, ngroup)
        def _(g):
            c0 = g * 2
            c1 = c0 + 1

            @pl.when(g > 0)
            def _():
                scatter_wait(c0 - 1, 1)

            gather_start(c1, 1)
            gather_wait(c0, 0)
            scatter_start(c0, 0)
            counts(c0)
            scatter_wait(c0, 0)

            @pl.when(g + 1 < ngroup)
            def _():
                gather_start(c0 + 2, 0)

            gather_wait(c1, 1)
            scatter_start(c1, 1)
            counts(c1)

        scatter_wait(nchunk - 1, 1)

        plsc.subcore_barrier()

        for c in range(nrow16 // CHUNK):
            pltpu.sync_copy(iota_hbm.at[pl.ds(c * CHUNK, CHUNK)], idxc_v)
            pltpu.sync_copy(cnt_v.at[pl.ds(c * CHUNK, CHUNK)],
                            cnt_sh.at[idxc_v], add=True)

        pltpu.sync_copy(acc_sh.at[pl.ds(sid * rpt, rpt)],
                        acc_out.at[cid, pl.ds(sid * rpt, rpt)])

        plsc.subcore_barrier()

        crows = nrow16 // NS
        pltpu.sync_copy(cnt_sh.at[pl.ds(sid * crows, crows)],
                        cnt_out.at[cid, pl.ds(sid * crows, crows)])

    return sc_kernel(xs, src, dst, z64, z16, iota)


def _tc_dense(p, cnt, x, wl, wr, b):
    n, d = x.shape
    blk = 2000

    def body(p_ref, c_ref, x_ref, wl_ref, wr_ref, b_ref, o_ref):
        c = jnp.clip(c_ref[0] + c_ref[1], 1.0)
        m0 = p_ref[0].astype(jnp.float32) / c
        m1 = p_ref[1].astype(jnp.float32) / c
        wl = wl_ref[...]
        o_ref[...] = (
            jnp.dot(m0, wl[:DH], preferred_element_type=jnp.float32,
                    precision=lax.Precision.HIGHEST)
            + jnp.dot(m1, wl[DH:], preferred_element_type=jnp.float32,
                      precision=lax.Precision.HIGHEST)
            + jnp.dot(x_ref[...], wr_ref[...], preferred_element_type=jnp.float32,
                      precision=lax.Precision.HIGHEST)
            + b_ref[...])

    return pl.pallas_call(
        body,
        grid=(n // blk,),
        in_specs=[
            pl.BlockSpec((NC, blk, DH), lambda i: (0, i, 0)),
            pl.BlockSpec((NC, blk, 1), lambda i: (0, i, 0)),
            pl.BlockSpec((blk, d), lambda i: (i, 0)),
            pl.BlockSpec((d, d), lambda i: (0, 0)),
            pl.BlockSpec((d, d), lambda i: (0, 0)),
            pl.BlockSpec((1, d), lambda i: (0, 0)),
        ],
        out_specs=pl.BlockSpec((blk, d), lambda i: (i, 0)),
        out_shape=jax.ShapeDtypeStruct((n, d), jnp.float32),
    )(p, cnt, x, wl, wr, b.reshape(1, d))


def kernel(x, edge_index, W_l, W_r, b):
    n, d = x.shape
    e = edge_index.shape[1]
    nchunk = -(-e // (NS * NSLOT * CHUNK)) * NSLOT
    ept = nchunk * CHUNK
    epad = ept * NS
    src = edge_index[0]
    dst = edge_index[1]
    if epad > e:
        pad = epad - e
        src = jnp.concatenate([src, jnp.zeros((pad,), jnp.int32)])
        dst = jnp.concatenate([dst, jnp.full((pad,), NACC - 1, jnp.int32)])
    src = src.reshape(NS, nchunk, CHUNK)
    dst = dst.reshape(NS, nchunk, CHUNK)

    xb = x.astype(jnp.bfloat16)
    xs = jnp.stack([xb[:, :DH], xb[:, DH:]])
    z64 = jnp.zeros((NACC // NS, DH), jnp.bfloat16)
    z16 = jnp.zeros((NACC // LANES, LANES), jnp.float32)
    iota = jnp.arange(NACC // LANES, dtype=jnp.int32)

    acc, cnt = _sc_aggregate(xs, src, dst, z64, z16, iota)
    cnt = cnt.reshape(NC, NACC, 1)
    return _tc_dense(acc, cnt, x, W_l, W_r, b)


# speedup vs baseline: 1.9810x; 1.2818x over previous
import dataclasses
import functools

import jax
import jax.numpy as jnp
from jax import lax
from jax.experimental import pallas as pl
from jax.experimental.pallas import tpu as pltpu
from jax.experimental.pallas import tpu_sc as plsc

NC = 2
NS = 16
LANES = 16
CHUNK = 128
NSLOT = 2
NACC = 10240
DH = 64


def _sc_aggregate(xs, src, dst, z64, z16, iota):
    nchunk = src.shape[1]
    ngroup = nchunk // NSLOT
    nrow16 = NACC // LANES

    mesh = plsc.VectorSubcoreMesh(core_axis_name="c", subcore_axis_name="s")

    cp = pltpu.CompilerParams()
    if "needs_layout_passes" in pltpu.CompilerParams.__dataclass_fields__:
        cp = dataclasses.replace(cp, needs_layout_passes=False)
    if "use_tc_tiling_on_sc" in pltpu.CompilerParams.__dataclass_fields__:
        cp = dataclasses.replace(cp, use_tc_tiling_on_sc=False)

    @functools.partial(
        pl.kernel,
        compiler_params=cp,
        out_type=[
            jax.ShapeDtypeStruct((NC, NACC, DH), jnp.bfloat16),
            jax.ShapeDtypeStruct((NC, nrow16, LANES), jnp.float32),
        ],
        mesh=mesh,
        scratch_types=[
            pltpu.VMEM((nchunk, CHUNK), jnp.int32),
            pltpu.VMEM((nchunk, CHUNK), jnp.int32),
            pltpu.VMEM((NSLOT, CHUNK, DH), jnp.bfloat16),
            pltpu.VMEM((nrow16, LANES), jnp.float32),
            pltpu.VMEM((CHUNK,), jnp.int32),
            pltpu.VMEM_SHARED((NACC, DH), jnp.bfloat16),
            pltpu.VMEM_SHARED((NACC, DH), jnp.bfloat16),
            pltpu.VMEM_SHARED((nrow16, LANES), jnp.float32),
            [pltpu.SemaphoreType.DMA] * NSLOT,
            [pltpu.SemaphoreType.DMA] * NSLOT,
        ],
    )
    def sc_kernel(xs_hbm, src_hbm, dst_hbm, z64_hbm, z16_hbm, iota_hbm,
                  acc_out, cnt_out, sidx_all, didx_all, rows_v, cnt_v,
                  idxc_v, acc_sh, x_sh, cnt_sh, sem_g, sem_s):
        cid = lax.axis_index("c")
        sid = lax.axis_index("s")
        rpt = NACC // NS
        xh = xs_hbm.at[cid]

        pltpu.sync_copy(z64_hbm, acc_sh.at[pl.ds(sid * rpt, rpt)])
        pltpu.sync_copy(z16_hbm, cnt_v)
        pltpu.sync_copy(src_hbm.at[sid], sidx_all)
        pltpu.sync_copy(dst_hbm.at[sid], didx_all)
        nxt = xh.shape[0] // NS
        pltpu.sync_copy(xh.at[pl.ds(sid * nxt, nxt)],
                        x_sh.at[pl.ds(sid * nxt, nxt)])

        @pl.when(sid == 0)
        def _():
            pltpu.sync_copy(z16_hbm, cnt_sh)

        plsc.subcore_barrier()

        ones = jnp.full((LANES,), 1.0, jnp.float32)
        four = jnp.full((LANES,), 4, jnp.int32)
        fifteen = jnp.full((LANES,), 15, jnp.int32)

        def gather_start(c, b):
            pltpu.async_copy(x_sh.at[sidx_all.at[c]], rows_v.at[b], sem_g[b])

        def gather_wait(c, b):
            pltpu.make_async_copy(x_sh.at[sidx_all.at[c]], rows_v.at[b],
                                  sem_g[b]).wait()

        def scatter_start(c, b):
            pltpu.async_copy(rows_v.at[b], acc_sh.at[didx_all.at[c]],
                             sem_s[b], add=True)

        def scatter_wait(c, b):
            pltpu.make_async_copy(rows_v.at[b], acc_sh.at[didx_all.at[c]],
                                  sem_s[b]).wait()

        def counts(c):
            @pl.when(lax.bitwise_and(c, 1) == cid)
            def _():
                for i in range(CHUNK // LANES):
                    dv = didx_all[c, pl.ds(i * LANES, LANES)]
                    row = lax.shift_right_logical(dv, four)
                    col = lax.bitwise_and(dv, fifteen)
                    plsc.addupdate_scatter(cnt_v, [row, col], ones)

        gather_start(0, 0)

        @pl.loop(0, ngroup)
        def _(g):
            c0 = g * 2
            c1 = c0 + 1

            @pl.when(g > 0)
            def _():
                scatter_wait(c0 - 1, 1)

            gather_start(c1, 1)
            gather_wait(c0, 0)
            scatter_start(c0, 0)
            counts(c0)
            scatter_wait(c0, 0)

            @pl.when(g + 1 < ngroup)
            def _():
                gather_start(c0 + 2, 0)

            gather_wait(c1, 1)
            scatter_start(c1, 1)
            counts(c1)

        scatter_wait(nchunk - 1, 1)

        plsc.subcore_barrier()

        for c in range(nrow16 // CHUNK):
            pltpu.sync_copy(iota_hbm.at[pl.ds(c * CHUNK, CHUNK)], idxc_v)
            pltpu.sync_copy(cnt_v.at[pl.ds(c * CHUNK, CHUNK)],
                            cnt_sh.at[idxc_v], add=True)

        pltpu.sync_copy(acc_sh.at[pl.ds(sid * rpt, rpt)],
                        acc_out.at[cid, pl.ds(sid * rpt, rpt)])

        plsc.subcore_barrier()

        crows = nrow16 // NS
        pltpu.sync_copy(cnt_sh.at[pl.ds(sid * crows, crows)],
                        cnt_out.at[cid, pl.ds(sid * crows, crows)])

    return sc_kernel(xs, src, dst, z64, z16, iota)


def _tc_dense(p, cnt, x, wl, wr, b):
    n, d = x.shape
    blk = 2000

    def body(p_ref, c_ref, x_ref, wl_ref, wr_ref, b_ref, o_ref):
        c = jnp.clip(c_ref[0] + c_ref[1], 1.0)
        m0 = p_ref[0].astype(jnp.float32) / c
        m1 = p_ref[1].astype(jnp.float32) / c
        wl = wl_ref[...]
        o_ref[...] = (
            jnp.dot(m0, wl[:DH], preferred_element_type=jnp.float32,
                    precision=lax.Precision.HIGHEST)
            + jnp.dot(m1, wl[DH:], preferred_element_type=jnp.float32,
                      precision=lax.Precision.HIGHEST)
            + jnp.dot(x_ref[...], wr_ref[...], preferred_element_type=jnp.float32,
                      precision=lax.Precision.HIGHEST)
            + b_ref[...])

    return pl.pallas_call(
        body,
        grid=(n // blk,),
        in_specs=[
            pl.BlockSpec((NC, blk, DH), lambda i: (0, i, 0)),
            pl.BlockSpec((NC, blk, 1), lambda i: (0, i, 0)),
            pl.BlockSpec((blk, d), lambda i: (i, 0)),
            pl.BlockSpec((d, d), lambda i: (0, 0)),
            pl.BlockSpec((d, d), lambda i: (0, 0)),
            pl.BlockSpec((1, d), lambda i: (0, 0)),
        ],
        out_specs=pl.BlockSpec((blk, d), lambda i: (i, 0)),
        out_shape=jax.ShapeDtypeStruct((n, d), jnp.float32),
    )(p, cnt, x, wl, wr, b.reshape(1, d))


def kernel(x, edge_index, W_l, W_r, b):
    n, d = x.shape
    e = edge_index.shape[1]
    nchunk = -(-e // (NS * NSLOT * CHUNK)) * NSLOT
    ept = nchunk * CHUNK
    epad = ept * NS
    src = edge_index[0]
    dst = edge_index[1]
    if epad > e:
        pad = epad - e
        src = jnp.concatenate([src, jnp.zeros((pad,), jnp.int32)])
        dst = jnp.concatenate([dst, jnp.full((pad,), NACC - 1, jnp.int32)])
    src = src.reshape(NS, nchunk, CHUNK)
    dst = dst.reshape(NS, nchunk, CHUNK)

    xb = x.astype(jnp.bfloat16)
    xs = jnp.stack([xb[:, :DH], xb[:, DH:]])
    z64 = jnp.zeros((NACC // NS, DH), jnp.bfloat16)
    z16 = jnp.zeros((NACC // LANES, LANES), jnp.float32)
    iota = jnp.arange(NACC // LANES, dtype=jnp.int32)

    acc, cnt = _sc_aggregate(xs, src, dst, z64, z16, iota)
    cnt = cnt.reshape(NC, NACC, 1)
    return _tc_dense(acc, cnt, x, W_l, W_r, b)


# speedup vs baseline: 1.9966x; 1.0079x over previous
import dataclasses
import functools

import jax
import jax.numpy as jnp
from jax import lax
from jax.experimental import pallas as pl
from jax.experimental.pallas import tpu as pltpu
from jax.experimental.pallas import tpu_sc as plsc

NC = 2
NS = 16
LANES = 16
CHUNK = 128
NSLOT = 2
NACC = 10240
DH = 64


def _sc_aggregate(xs, src, dst, z64, z16, iota):
    nchunk = src.shape[1]
    ngroup = nchunk // NSLOT
    nrow16 = NACC // LANES

    mesh = plsc.VectorSubcoreMesh(core_axis_name="c", subcore_axis_name="s")

    cp = pltpu.CompilerParams()
    if "needs_layout_passes" in pltpu.CompilerParams.__dataclass_fields__:
        cp = dataclasses.replace(cp, needs_layout_passes=False)
    if "use_tc_tiling_on_sc" in pltpu.CompilerParams.__dataclass_fields__:
        cp = dataclasses.replace(cp, use_tc_tiling_on_sc=False)

    @functools.partial(
        pl.kernel,
        compiler_params=cp,
        out_type=[
            jax.ShapeDtypeStruct((NC, NACC, DH), jnp.bfloat16),
            jax.ShapeDtypeStruct((NC, nrow16, LANES), jnp.float32),
        ],
        mesh=mesh,
        scratch_types=[
            pltpu.VMEM((nchunk, CHUNK), jnp.int32),
            pltpu.VMEM((nchunk, CHUNK), jnp.int32),
            pltpu.VMEM((NSLOT, CHUNK, DH), jnp.bfloat16),
            pltpu.VMEM((nrow16, LANES), jnp.float32),
            pltpu.VMEM((CHUNK,), jnp.int32),
            pltpu.VMEM_SHARED((NACC, DH), jnp.bfloat16),
            pltpu.VMEM_SHARED((NACC, DH), jnp.bfloat16),
            pltpu.VMEM_SHARED((nrow16, LANES), jnp.float32),
            [pltpu.SemaphoreType.DMA] * NSLOT,
            [pltpu.SemaphoreType.DMA] * NSLOT,
        ],
    )
    def sc_kernel(xs_hbm, src_hbm, dst_hbm, z64_hbm, z16_hbm, iota_hbm,
                  acc_out, cnt_out, sidx_all, didx_all, rows_v, cnt_v,
                  idxc_v, acc_sh, x_sh, cnt_sh, sem_g, sem_s):
        cid = lax.axis_index("c")
        sid = lax.axis_index("s")
        rpt = NACC // NS
        xh = xs_hbm.at[cid]

        pltpu.sync_copy(z64_hbm, acc_sh.at[pl.ds(sid * rpt, rpt)])
        pltpu.sync_copy(z16_hbm, cnt_v)
        pltpu.sync_copy(src_hbm.at[sid], sidx_all)
        pltpu.sync_copy(dst_hbm.at[sid], didx_all)
        nxt = xh.shape[0] // NS
        pltpu.sync_copy(xh.at[pl.ds(sid * nxt, nxt)],
                        x_sh.at[pl.ds(sid * nxt, nxt)])

        @pl.when(sid == 0)
        def _():
            pltpu.sync_copy(z16_hbm, cnt_sh)

        plsc.subcore_barrier()

        ones = jnp.full((LANES,), 1.0, jnp.float32)
        four = jnp.full((LANES,), 4, jnp.int32)
        fifteen = jnp.full((LANES,), 15, jnp.int32)

        def gather_start(c, b):
            pltpu.async_copy(x_sh.at[sidx_all.at[c]], rows_v.at[b], sem_g[b])

        def gather_wait(c, b):
            pltpu.make_async_copy(x_sh.at[sidx_all.at[c]], rows_v.at[b],
                                  sem_g[b]).wait()

        def scatter_start(c, b):
            pltpu.async_copy(rows_v.at[b], acc_sh.at[didx_all.at[c]],
                             sem_s[b], add=True)

        def scatter_wait(c, b):
            pltpu.make_async_copy(rows_v.at[b], acc_sh.at[didx_all.at[c]],
                                  sem_s[b]).wait()

        def counts(c):
            @pl.when(lax.bitwise_and(c, 1) == cid)
            def _():
                for i in range(CHUNK // LANES):
                    dv = didx_all[c, pl.ds(i * LANES, LANES)]
                    row = lax.shift_right_logical(dv, four)
                    col = lax.bitwise_and(dv, fifteen)
                    plsc.addupdate_scatter(cnt_v, [row, col], ones)

        gather_start(0, 0)

        @pl.loop(0, ngroup)
        def _(g):
            c0 = g * 2
            c1 = c0 + 1

            @pl.when(g > 0)
            def _():
                scatter_wait(c0 - 1, 1)

            gather_start(c1, 1)
            gather_wait(c0, 0)
            scatter_start(c0, 0)
            counts(c0)
            scatter_wait(c0, 0)

            @pl.when(g + 1 < ngroup)
            def _():
                gather_start(c0 + 2, 0)

            gather_wait(c1, 1)
            scatter_start(c1, 1)
            counts(c1)

        scatter_wait(nchunk - 1, 1)

        plsc.subcore_barrier()

        for c in range(nrow16 // CHUNK):
            pltpu.sync_copy(iota_hbm.at[pl.ds(c * CHUNK, CHUNK)], idxc_v)
            pltpu.sync_copy(cnt_v.at[pl.ds(c * CHUNK, CHUNK)],
                            cnt_sh.at[idxc_v], add=True)

        pltpu.sync_copy(acc_sh.at[pl.ds(sid * rpt, rpt)],
                        acc_out.at[cid, pl.ds(sid * rpt, rpt)])

        plsc.subcore_barrier()

        crows = nrow16 // NS
        pltpu.sync_copy(cnt_sh.at[pl.ds(sid * crows, crows)],
                        cnt_out.at[cid, pl.ds(sid * crows, crows)])

    return sc_kernel(xs, src, dst, z64, z16, iota)


def _tc_root(x, wr, b):
    n, d = x.shape
    blk = 2000

    def body(x_ref, wr_ref, b_ref, o_ref):
        o_ref[...] = jnp.dot(
            x_ref[...], wr_ref[...], preferred_element_type=jnp.float32,
            precision=lax.Precision.HIGHEST) + b_ref[...]

    return pl.pallas_call(
        body,
        grid=(n // blk,),
        in_specs=[
            pl.BlockSpec((blk, d), lambda i: (i, 0)),
            pl.BlockSpec((d, d), lambda i: (0, 0)),
            pl.BlockSpec((1, d), lambda i: (0, 0)),
        ],
        out_specs=pl.BlockSpec((blk, d), lambda i: (i, 0)),
        out_shape=jax.ShapeDtypeStruct((n, d), jnp.float32),
    )(x, wr, b.reshape(1, d))


def _tc_dense(p, cnt, r, wl):
    n, d = r.shape
    blk = 2000

    def body(p_ref, c_ref, r_ref, wl_ref, o_ref):
        c = jnp.clip(c_ref[0] + c_ref[1], 1.0)
        m0 = p_ref[0].astype(jnp.float32) / c
        m1 = p_ref[1].astype(jnp.float32) / c
        wl = wl_ref[...]
        o_ref[...] = (
            jnp.dot(m0, wl[:DH], preferred_element_type=jnp.float32,
                    precision=lax.Precision.HIGHEST)
            + jnp.dot(m1, wl[DH:], preferred_element_type=jnp.float32,
                      precision=lax.Precision.HIGHEST)
            + r_ref[...])

    return pl.pallas_call(
        body,
        grid=(n // blk,),
        in_specs=[
            pl.BlockSpec((NC, blk, DH), lambda i: (0, i, 0)),
            pl.BlockSpec((NC, blk, 1), lambda i: (0, i, 0)),
            pl.BlockSpec((blk, d), lambda i: (i, 0)),
            pl.BlockSpec((d, d), lambda i: (0, 0)),
        ],
        out_specs=pl.BlockSpec((blk, d), lambda i: (i, 0)),
        out_shape=jax.ShapeDtypeStruct((n, d), jnp.float32),
    )(p, cnt, r, wl)


def kernel(x, edge_index, W_l, W_r, b):
    n, d = x.shape
    e = edge_index.shape[1]
    nchunk = -(-e // (NS * NSLOT * CHUNK)) * NSLOT
    ept = nchunk * CHUNK
    epad = ept * NS
    src = edge_index[0]
    dst = edge_index[1]
    if epad > e:
        pad = epad - e
        src = jnp.concatenate([src, jnp.zeros((pad,), jnp.int32)])
        dst = jnp.concatenate([dst, jnp.full((pad,), NACC - 1, jnp.int32)])
    src = src.reshape(NS, nchunk, CHUNK)
    dst = dst.reshape(NS, nchunk, CHUNK)

    xb = x.astype(jnp.bfloat16)
    xs = jnp.stack([xb[:, :DH], xb[:, DH:]])
    z64 = jnp.zeros((NACC // NS, DH), jnp.bfloat16)
    z16 = jnp.zeros((NACC // LANES, LANES), jnp.float32)
    iota = jnp.arange(NACC // LANES, dtype=jnp.int32)

    acc, cnt = _sc_aggregate(xs, src, dst, z64, z16, iota)
    r = _tc_root(x, W_r, b)
    cnt = cnt.reshape(NC, NACC, 1)
    return _tc_dense(acc, cnt, r, W_l)


# speedup vs baseline: 2.0371x; 1.0203x over previous
import dataclasses
import functools

import jax
import jax.numpy as jnp
from jax import lax
from jax.experimental import pallas as pl
from jax.experimental.pallas import tpu as pltpu
from jax.experimental.pallas import tpu_sc as plsc

NC = 2
NS = 16
LANES = 16
CHUNK = 128
NSLOT = 2
NACC = 10240
DH = 64


def _sc_aggregate(xs, src, dst, z64, z16, iota):
    nchunk = src.shape[1]
    ngroup = nchunk // NSLOT
    nrow16 = NACC // LANES

    mesh = plsc.VectorSubcoreMesh(core_axis_name="c", subcore_axis_name="s")

    cp = pltpu.CompilerParams()
    if "needs_layout_passes" in pltpu.CompilerParams.__dataclass_fields__:
        cp = dataclasses.replace(cp, needs_layout_passes=False)
    if "use_tc_tiling_on_sc" in pltpu.CompilerParams.__dataclass_fields__:
        cp = dataclasses.replace(cp, use_tc_tiling_on_sc=False)

    @functools.partial(
        pl.kernel,
        compiler_params=cp,
        out_type=[
            jax.ShapeDtypeStruct((NC, NACC, DH), jnp.bfloat16),
            jax.ShapeDtypeStruct((NC, nrow16, LANES), jnp.float32),
        ],
        mesh=mesh,
        scratch_types=[
            pltpu.VMEM((nchunk, CHUNK), jnp.int32),
            pltpu.VMEM((nchunk, CHUNK), jnp.int32),
            pltpu.VMEM((NSLOT, CHUNK, DH), jnp.bfloat16),
            pltpu.VMEM((nrow16, LANES), jnp.float32),
            pltpu.VMEM((CHUNK,), jnp.int32),
            pltpu.VMEM_SHARED((NACC, DH), jnp.bfloat16),
            pltpu.VMEM_SHARED((NACC, DH), jnp.bfloat16),
            pltpu.VMEM_SHARED((nrow16, LANES), jnp.float32),
            [pltpu.SemaphoreType.DMA] * NSLOT,
            [pltpu.SemaphoreType.DMA] * NSLOT,
        ],
    )
    def sc_kernel(xs_hbm, src_hbm, dst_hbm, z64_hbm, z16_hbm, iota_hbm,
                  acc_out, cnt_out, sidx_all, didx_all, rows_v, cnt_v,
                  idxc_v, acc_sh, x_sh, cnt_sh, sem_g, sem_s):
        cid = lax.axis_index("c")
        sid = lax.axis_index("s")
        rpt = NACC // NS
        xh = xs_hbm.at[cid]

        pltpu.sync_copy(z64_hbm, acc_sh.at[pl.ds(sid * rpt, rpt)])
        pltpu.sync_copy(z16_hbm, cnt_v)
        pltpu.sync_copy(src_hbm.at[sid], sidx_all)
        pltpu.sync_copy(dst_hbm.at[sid], didx_all)
        nxt = xh.shape[0] // NS
        pltpu.sync_copy(xh.at[pl.ds(sid * nxt, nxt)],
                        x_sh.at[pl.ds(sid * nxt, nxt)])

        @pl.when(sid == 0)
        def _():
            pltpu.sync_copy(z16_hbm, cnt_sh)

        plsc.subcore_barrier()

        ones = jnp.full((LANES,), 1.0, jnp.float32)
        four = jnp.full((LANES,), 4, jnp.int32)
        fifteen = jnp.full((LANES,), 15, jnp.int32)

        def gather_start(c, b):
            pltpu.async_copy(x_sh.at[sidx_all.at[c]], rows_v.at[b], sem_g[b])

        def gather_wait(c, b):
            pltpu.make_async_copy(x_sh.at[sidx_all.at[c]], rows_v.at[b],
                                  sem_g[b]).wait()

        def scatter_start(c, b):
            pltpu.async_copy(rows_v.at[b], acc_sh.at[didx_all.at[c]],
                             sem_s[b], add=True)

        def scatter_wait(c, b):
            pltpu.make_async_copy(rows_v.at[b], acc_sh.at[didx_all.at[c]],
                                  sem_s[b]).wait()

        def counts(c):
            @pl.when(lax.bitwise_and(c, 1) == cid)
            def _():
                for i in range(CHUNK // LANES):
                    dv = didx_all[c, pl.ds(i * LANES, LANES)]
                    row = lax.shift_right_logical(dv, four)
                    col = lax.bitwise_and(dv, fifteen)
                    plsc.addupdate_scatter(cnt_v, [row, col], ones)

        gather_start(0, 0)

        @pl.loop(0, ngroup)
        def _(g):
            c0 = g * 2
            c1 = c0 + 1

            @pl.when(g > 0)
            def _():
                scatter_wait(c0 - 1, 1)

            gather_start(c1, 1)
            gather_wait(c0, 0)
            scatter_start(c0, 0)
            counts(c0)
            scatter_wait(c0, 0)

            @pl.when(g + 1 < ngroup)
            def _():
                gather_start(c0 + 2, 0)

            gather_wait(c1, 1)
            scatter_start(c1, 1)
            counts(c1)

        scatter_wait(nchunk - 1, 1)

        plsc.subcore_barrier()

        for c in range(nrow16 // CHUNK):
            pltpu.sync_copy(iota_hbm.at[pl.ds(c * CHUNK, CHUNK)], idxc_v)
            pltpu.sync_copy(cnt_v.at[pl.ds(c * CHUNK, CHUNK)],
                            cnt_sh.at[idxc_v], add=True)

        pltpu.sync_copy(acc_sh.at[pl.ds(sid * rpt, rpt)],
                        acc_out.at[cid, pl.ds(sid * rpt, rpt)])

        plsc.subcore_barrier()

        crows = nrow16 // NS
        pltpu.sync_copy(cnt_sh.at[pl.ds(sid * crows, crows)],
                        cnt_out.at[cid, pl.ds(sid * crows, crows)])

    return sc_kernel(xs, src, dst, z64, z16, iota)


def _tc_root(x, wr, b):
    n, d = x.shape
    blk = 2000

    def body(x_ref, wr_ref, b_ref, o_ref):
        o_ref[...] = jnp.dot(
            x_ref[...], wr_ref[...], preferred_element_type=jnp.float32) + b_ref[...]

    return pl.pallas_call(
        body,
        grid=(n // blk,),
        in_specs=[
            pl.BlockSpec((blk, d), lambda i: (i, 0)),
            pl.BlockSpec((d, d), lambda i: (0, 0)),
            pl.BlockSpec((1, d), lambda i: (0, 0)),
        ],
        out_specs=pl.BlockSpec((blk, d), lambda i: (i, 0)),
        out_shape=jax.ShapeDtypeStruct((n, d), jnp.float32),
    )(x, wr, b.reshape(1, d))


def _tc_dense(p, cnt, r, wl):
    n, d = r.shape
    blk = 2000

    def body(p_ref, c_ref, r_ref, wl_ref, o_ref):
        c = jnp.clip(c_ref[0] + c_ref[1], 1.0)
        m0 = p_ref[0].astype(jnp.float32) / c
        m1 = p_ref[1].astype(jnp.float32) / c
        wl = wl_ref[...]
        o_ref[...] = (
            jnp.dot(m0, wl[:DH], preferred_element_type=jnp.float32)
            + jnp.dot(m1, wl[DH:], preferred_element_type=jnp.float32)
            + r_ref[...])

    return pl.pallas_call(
        body,
        grid=(n // blk,),
        in_specs=[
            pl.BlockSpec((NC, blk, DH), lambda i: (0, i, 0)),
            pl.BlockSpec((NC, blk, 1), lambda i: (0, i, 0)),
            pl.BlockSpec((blk, d), lambda i: (i, 0)),
            pl.BlockSpec((d, d), lambda i: (0, 0)),
        ],
        out_specs=pl.BlockSpec((blk, d), lambda i: (i, 0)),
        out_shape=jax.ShapeDtypeStruct((n, d), jnp.float32),
    )(p, cnt, r, wl)


def kernel(x, edge_index, W_l, W_r, b):
    n, d = x.shape
    e = edge_index.shape[1]
    nchunk = -(-e // (NS * NSLOT * CHUNK)) * NSLOT
    ept = nchunk * CHUNK
    epad = ept * NS
    src = edge_index[0]
    dst = edge_index[1]
    if epad > e:
        pad = epad - e
        src = jnp.concatenate([src, jnp.zeros((pad,), jnp.int32)])
        dst = jnp.concatenate([dst, jnp.full((pad,), NACC - 1, jnp.int32)])
    src = src.reshape(NS, nchunk, CHUNK)
    dst = dst.reshape(NS, nchunk, CHUNK)

    xb = x.astype(jnp.bfloat16)
    xs = jnp.stack([xb[:, :DH], xb[:, DH:]])
    z64 = jnp.zeros((NACC // NS, DH), jnp.bfloat16)
    z16 = jnp.zeros((NACC // LANES, LANES), jnp.float32)
    iota = jnp.arange(NACC // LANES, dtype=jnp.int32)

    acc, cnt = _sc_aggregate(xs, src, dst, z64, z16, iota)
    r = _tc_root(x, W_r, b)
    cnt = cnt.reshape(NC, NACC, 1)
    return _tc_dense(acc, cnt, r, W_l)


# speedup vs baseline: 2.1054x; 1.0335x over previous
import dataclasses
import functools

import jax
import jax.numpy as jnp
from jax import lax
from jax.experimental import pallas as pl
from jax.experimental.pallas import tpu as pltpu
from jax.experimental.pallas import tpu_sc as plsc

NC = 2
NS = 16
LANES = 16
CHUNK = 128
NSLOT = 2
NACC = 10240
DH = 64


def _sc_aggregate(xs, src, dst, z64, z16, iota):
    nchunk = src.shape[1]
    ngroup = nchunk // NSLOT
    nrow16 = NACC // LANES

    mesh = plsc.VectorSubcoreMesh(core_axis_name="c", subcore_axis_name="s")

    cp = pltpu.CompilerParams()
    if "needs_layout_passes" in pltpu.CompilerParams.__dataclass_fields__:
        cp = dataclasses.replace(cp, needs_layout_passes=False)
    if "use_tc_tiling_on_sc" in pltpu.CompilerParams.__dataclass_fields__:
        cp = dataclasses.replace(cp, use_tc_tiling_on_sc=False)

    @functools.partial(
        pl.kernel,
        compiler_params=cp,
        out_type=[
            jax.ShapeDtypeStruct((NC, NACC, DH), jnp.bfloat16),
            jax.ShapeDtypeStruct((NC, nrow16, LANES), jnp.float32),
        ],
        mesh=mesh,
        scratch_types=[
            pltpu.VMEM((nchunk, CHUNK), jnp.int32),
            pltpu.VMEM((nchunk, CHUNK), jnp.int32),
            pltpu.VMEM((NSLOT, CHUNK, DH), jnp.bfloat16),
            pltpu.VMEM((nrow16, LANES), jnp.float32),
            pltpu.VMEM((CHUNK,), jnp.int32),
            pltpu.VMEM_SHARED((NACC, DH), jnp.bfloat16),
            pltpu.VMEM_SHARED((NACC, DH), jnp.bfloat16),
            pltpu.VMEM_SHARED((nrow16, LANES), jnp.float32),
            [pltpu.SemaphoreType.DMA] * NSLOT,
            [pltpu.SemaphoreType.DMA] * NSLOT,
        ],
    )
    def sc_kernel(xb_hbm, src_hbm, dst_hbm, z64_hbm, z16_hbm, iota_hbm,
                  acc_out, cnt_out, sidx_all, didx_all, rows_v, cnt_v,
                  idxc_v, acc_sh, x_sh, cnt_sh, sem_g, sem_s):
        cid = lax.axis_index("c")
        sid = lax.axis_index("s")
        rpt = NACC // NS

        pltpu.sync_copy(z64_hbm, acc_sh.at[pl.ds(sid * rpt, rpt)])
        pltpu.sync_copy(z16_hbm, cnt_v)
        pltpu.sync_copy(src_hbm.at[sid], sidx_all)
        pltpu.sync_copy(dst_hbm.at[sid], didx_all)
        nxt = xb_hbm.shape[0] // NS
        pltpu.sync_copy(xb_hbm.at[pl.ds(sid * nxt, nxt),
                                  pl.ds(cid * DH, DH)],
                        x_sh.at[pl.ds(sid * nxt, nxt)])

        @pl.when(sid == 0)
        def _():
            pltpu.sync_copy(z16_hbm, cnt_sh)

        plsc.subcore_barrier()

        ones = jnp.full((LANES,), 1.0, jnp.float32)
        four = jnp.full((LANES,), 4, jnp.int32)
        fifteen = jnp.full((LANES,), 15, jnp.int32)

        def gather_start(c, b):
            pltpu.async_copy(x_sh.at[sidx_all.at[c]], rows_v.at[b], sem_g[b])

        def gather_wait(c, b):
            pltpu.make_async_copy(x_sh.at[sidx_all.at[c]], rows_v.at[b],
                                  sem_g[b]).wait()

        def scatter_start(c, b):
            pltpu.async_copy(rows_v.at[b], acc_sh.at[didx_all.at[c]],
                             sem_s[b], add=True)

        def scatter_wait(c, b):
            pltpu.make_async_copy(rows_v.at[b], acc_sh.at[didx_all.at[c]],
                                  sem_s[b]).wait()

        def counts(c):
            @pl.when(lax.bitwise_and(c, 1) == cid)
            def _():
                for i in range(CHUNK // LANES):
                    dv = didx_all[c, pl.ds(i * LANES, LANES)]
                    row = lax.shift_right_logical(dv, four)
                    col = lax.bitwise_and(dv, fifteen)
                    plsc.addupdate_scatter(cnt_v, [row, col], ones)

        gather_start(0, 0)

        @pl.loop(0, ngroup)
        def _(g):
            c0 = g * 2
            c1 = c0 + 1

            @pl.when(g > 0)
            def _():
                scatter_wait(c0 - 1, 1)

            gather_start(c1, 1)
            gather_wait(c0, 0)
            scatter_start(c0, 0)
            counts(c0)
            scatter_wait(c0, 0)

            @pl.when(g + 1 < ngroup)
            def _():
                gather_start(c0 + 2, 0)

            gather_wait(c1, 1)
            scatter_start(c1, 1)
            counts(c1)

        scatter_wait(nchunk - 1, 1)

        plsc.subcore_barrier()

        for c in range(nrow16 // CHUNK):
            pltpu.sync_copy(iota_hbm.at[pl.ds(c * CHUNK, CHUNK)], idxc_v)
            pltpu.sync_copy(cnt_v.at[pl.ds(c * CHUNK, CHUNK)],
                            cnt_sh.at[idxc_v], add=True)

        pltpu.sync_copy(acc_sh.at[pl.ds(sid * rpt, rpt)],
                        acc_out.at[cid, pl.ds(sid * rpt, rpt)])

        plsc.subcore_barrier()

        crows = nrow16 // NS
        pltpu.sync_copy(cnt_sh.at[pl.ds(sid * crows, crows)],
                        cnt_out.at[cid, pl.ds(sid * crows, crows)])

    return sc_kernel(xs, src, dst, z64, z16, iota)


def _tc_root(x, wr, b):
    n, d = x.shape
    blk = 2000

    def body(x_ref, wr_ref, b_ref, o_ref):
        o_ref[...] = jnp.dot(
            x_ref[...], wr_ref[...], preferred_element_type=jnp.float32) + b_ref[...]

    return pl.pallas_call(
        body,
        grid=(n // blk,),
        in_specs=[
            pl.BlockSpec((blk, d), lambda i: (i, 0)),
            pl.BlockSpec((d, d), lambda i: (0, 0)),
            pl.BlockSpec((1, d), lambda i: (0, 0)),
        ],
        out_specs=pl.BlockSpec((blk, d), lambda i: (i, 0)),
        out_shape=jax.ShapeDtypeStruct((n, d), jnp.float32),
    )(x, wr, b.reshape(1, d))


def _tc_dense(p, cnt, r, wl):
    n, d = r.shape
    blk = 2000

    def body(p_ref, c_ref, r_ref, wl_ref, o_ref):
        c = jnp.clip(c_ref[0] + c_ref[1], 1.0)
        m0 = p_ref[0].astype(jnp.float32) / c
        m1 = p_ref[1].astype(jnp.float32) / c
        wl = wl_ref[...]
        o_ref[...] = (
            jnp.dot(m0, wl[:DH], preferred_element_type=jnp.float32)
            + jnp.dot(m1, wl[DH:], preferred_element_type=jnp.float32)
            + r_ref[...])

    return pl.pallas_call(
        body,
        grid=(n // blk,),
        in_specs=[
            pl.BlockSpec((NC, blk, DH), lambda i: (0, i, 0)),
            pl.BlockSpec((NC, blk, 1), lambda i: (0, i, 0)),
            pl.BlockSpec((blk, d), lambda i: (i, 0)),
            pl.BlockSpec((d, d), lambda i: (0, 0)),
        ],
        out_specs=pl.BlockSpec((blk, d), lambda i: (i, 0)),
        out_shape=jax.ShapeDtypeStruct((n, d), jnp.float32),
    )(p, cnt, r, wl)


def kernel(x, edge_index, W_l, W_r, b):
    n, d = x.shape
    e = edge_index.shape[1]
    nchunk = -(-e // (NS * NSLOT * CHUNK)) * NSLOT
    ept = nchunk * CHUNK
    epad = ept * NS
    src = edge_index[0]
    dst = edge_index[1]
    if epad > e:
        pad = epad - e
        src = jnp.concatenate([src, jnp.zeros((pad,), jnp.int32)])
        dst = jnp.concatenate([dst, jnp.full((pad,), NACC - 1, jnp.int32)])
    src = src.reshape(NS, nchunk, CHUNK)
    dst = dst.reshape(NS, nchunk, CHUNK)

    xb = x.astype(jnp.bfloat16)
    z64 = jnp.zeros((NACC // NS, DH), jnp.bfloat16)
    z16 = jnp.zeros((NACC // LANES, LANES), jnp.float32)
    iota = jnp.arange(NACC // LANES, dtype=jnp.int32)

    acc, cnt = _sc_aggregate(xb, src, dst, z64, z16, iota)
    r = _tc_root(x, W_r, b)
    cnt = cnt.reshape(NC, NACC, 1)
    return _tc_dense(acc, cnt, r, W_l)


# speedup vs baseline: 2.1268x; 1.0102x over previous
import dataclasses
import functools

import jax
import jax.numpy as jnp
from jax import lax
from jax.experimental import pallas as pl
from jax.experimental.pallas import tpu as pltpu
from jax.experimental.pallas import tpu_sc as plsc

NC = 2
NS = 16
LANES = 16
CHUNK = 128
NSLOT = 2
NACC = 10240
DH = 64


def _sc_aggregate(xs, src, dst, z64, z16, iota):
    nchunk = src.shape[1]
    ngroup = nchunk // NSLOT
    nrow16 = NACC // LANES

    mesh = plsc.VectorSubcoreMesh(core_axis_name="c", subcore_axis_name="s")

    cp = pltpu.CompilerParams()
    if "needs_layout_passes" in pltpu.CompilerParams.__dataclass_fields__:
        cp = dataclasses.replace(cp, needs_layout_passes=False)
    if "use_tc_tiling_on_sc" in pltpu.CompilerParams.__dataclass_fields__:
        cp = dataclasses.replace(cp, use_tc_tiling_on_sc=False)

    @functools.partial(
        pl.kernel,
        compiler_params=cp,
        out_type=[
            jax.ShapeDtypeStruct((NC, NACC, DH), jnp.bfloat16),
            jax.ShapeDtypeStruct((NC, nrow16, LANES), jnp.float32),
        ],
        mesh=mesh,
        scratch_types=[
            pltpu.VMEM((nchunk, CHUNK), jnp.int32),
            pltpu.VMEM((nchunk, CHUNK), jnp.int32),
            pltpu.VMEM((NSLOT, CHUNK, DH), jnp.bfloat16),
            pltpu.VMEM((nrow16, LANES), jnp.float32),
            pltpu.VMEM((CHUNK,), jnp.int32),
            pltpu.VMEM_SHARED((NACC, DH), jnp.bfloat16),
            pltpu.VMEM_SHARED((NACC, DH), jnp.bfloat16),
            pltpu.VMEM_SHARED((nrow16, LANES), jnp.float32),
            [pltpu.SemaphoreType.DMA] * NSLOT,
            [pltpu.SemaphoreType.DMA] * NSLOT,
        ],
    )
    def sc_kernel(xb_hbm, src_hbm, dst_hbm, z64_hbm, z16_hbm, iota_hbm,
                  acc_out, cnt_out, sidx_all, didx_all, rows_v, cnt_v,
                  idxc_v, acc_sh, x_sh, cnt_sh, sem_g, sem_s):
        cid = lax.axis_index("c")
        sid = lax.axis_index("s")
        rpt = NACC // NS

        pltpu.sync_copy(z64_hbm, acc_sh.at[pl.ds(sid * rpt, rpt)])
        pltpu.sync_copy(z16_hbm, cnt_v)
        pltpu.sync_copy(src_hbm.at[sid], sidx_all)
        pltpu.sync_copy(dst_hbm.at[sid], didx_all)
        nxt = xb_hbm.shape[0] // NS
        pltpu.sync_copy(xb_hbm.at[pl.ds(sid * nxt, nxt),
                                  pl.ds(cid * DH, DH)],
                        x_sh.at[pl.ds(sid * nxt, nxt)])

        @pl.when(sid == 0)
        def _():
            pltpu.sync_copy(z16_hbm, cnt_sh)

        plsc.subcore_barrier()

        ones = jnp.full((LANES,), 1.0, jnp.float32)
        four = jnp.full((LANES,), 4, jnp.int32)
        fifteen = jnp.full((LANES,), 15, jnp.int32)

        def gather_start(c, b):
            pltpu.async_copy(x_sh.at[sidx_all.at[c]], rows_v.at[b], sem_g[b])

        def gather_wait(c, b):
            pltpu.make_async_copy(x_sh.at[sidx_all.at[c]], rows_v.at[b],
                                  sem_g[b]).wait()

        def scatter_start(c, b):
            pltpu.async_copy(rows_v.at[b], acc_sh.at[didx_all.at[c]],
                             sem_s[b], add=True)

        def scatter_wait(c, b):
            pltpu.make_async_copy(rows_v.at[b], acc_sh.at[didx_all.at[c]],
                                  sem_s[b]).wait()

        def counts(c):
            @pl.when(lax.bitwise_and(c, 1) == cid)
            def _():
                for i in range(CHUNK // LANES):
                    dv = didx_all[c, pl.ds(i * LANES, LANES)]
                    row = lax.shift_right_logical(dv, four)
                    col = lax.bitwise_and(dv, fifteen)
                    plsc.addupdate_scatter(cnt_v, [row, col], ones)

        gather_start(0, 0)

        @pl.loop(0, ngroup)
        def _(g):
            c0 = g * 2
            c1 = c0 + 1

            @pl.when(g > 0)
            def _():
                scatter_wait(c0 - 1, 1)

            gather_start(c1, 1)
            gather_wait(c0, 0)
            scatter_start(c0, 0)
            counts(c0)
            scatter_wait(c0, 0)

            @pl.when(g + 1 < ngroup)
            def _():
                gather_start(c0 + 2, 0)

            gather_wait(c1, 1)
            scatter_start(c1, 1)
            counts(c1)

        scatter_wait(nchunk - 1, 1)

        plsc.subcore_barrier()

        for c in range(nrow16 // CHUNK):
            pltpu.sync_copy(iota_hbm.at[pl.ds(c * CHUNK, CHUNK)], idxc_v)
            pltpu.sync_copy(cnt_v.at[pl.ds(c * CHUNK, CHUNK)],
                            cnt_sh.at[idxc_v], add=True)

        pltpu.sync_copy(acc_sh.at[pl.ds(sid * rpt, rpt)],
                        acc_out.at[cid, pl.ds(sid * rpt, rpt)])

        plsc.subcore_barrier()

        crows = nrow16 // NS
        pltpu.sync_copy(cnt_sh.at[pl.ds(sid * crows, crows)],
                        cnt_out.at[cid, pl.ds(sid * crows, crows)])

    return sc_kernel(xs, src, dst, z64, z16, iota)


def _tc_root(x, wr, b):
    n, d = x.shape
    blk = 2000

    def body(x_ref, wr_ref, b_ref, o_ref):
        o_ref[...] = jnp.dot(
            x_ref[...], wr_ref[...], preferred_element_type=jnp.float32) + b_ref[...]

    return pl.pallas_call(
        body,
        grid=(n // blk,),
        in_specs=[
            pl.BlockSpec((blk, d), lambda i: (i, 0)),
            pl.BlockSpec((d, d), lambda i: (0, 0)),
            pl.BlockSpec((1, d), lambda i: (0, 0)),
        ],
        out_specs=pl.BlockSpec((blk, d), lambda i: (i, 0)),
        out_shape=jax.ShapeDtypeStruct((n, d), jnp.float32),
    )(x, wr, b.reshape(1, d))


def _tc_dense(p, cnt, r, wl):
    n, d = r.shape
    blk = 2000

    def body(p_ref, c_ref, r_ref, wl_ref, o_ref):
        c = jnp.clip(c_ref[0] + c_ref[1], 1.0)
        m0 = p_ref[0].astype(jnp.float32) / c
        m1 = p_ref[1].astype(jnp.float32) / c
        wl = wl_ref[...]
        o_ref[...] = (
            jnp.dot(m0, wl[:DH], preferred_element_type=jnp.float32)
            + jnp.dot(m1, wl[DH:], preferred_element_type=jnp.float32)
            + r_ref[...])

    return pl.pallas_call(
        body,
        grid=(n // blk,),
        in_specs=[
            pl.BlockSpec((NC, blk, DH), lambda i: (0, i, 0)),
            pl.BlockSpec((NC, blk, 1), lambda i: (0, i, 0)),
            pl.BlockSpec((blk, d), lambda i: (i, 0)),
            pl.BlockSpec((d, d), lambda i: (0, 0)),
        ],
        out_specs=pl.BlockSpec((blk, d), lambda i: (i, 0)),
        out_shape=jax.ShapeDtypeStruct((n, d), jnp.float32),
    )(p, cnt, r, wl)


def kernel(x, edge_index, W_l, W_r, b):
    n, d = x.shape
    e = edge_index.shape[1]
    nchunk = -(-e // (NS * NSLOT * CHUNK)) * NSLOT
    ept = nchunk * CHUNK
    epad = ept * NS
    src = edge_index[0]
    dst = edge_index[1]
    if epad > e:
        pad = epad - e
        src = jnp.concatenate([src, jnp.zeros((pad,), jnp.int32)])
        dst = jnp.concatenate([dst, jnp.full((pad,), NACC - 1, jnp.int32)])
    src = src.reshape(NS, nchunk, CHUNK)
    dst = dst.reshape(NS, nchunk, CHUNK)

    xb = x.astype(jnp.bfloat16)
    z64 = jnp.zeros((NACC // NS, DH), jnp.bfloat16)
    z16 = jnp.zeros((NACC // LANES, LANES), jnp.float32)
    iota = jnp.arange(NACC // LANES, dtype=jnp.int32)

    acc, cnt = _sc_aggregate(xb, src, dst, z64, z16, iota)
    r = _tc_root(xb, W_r.astype(jnp.bfloat16), b)
    cnt = cnt.reshape(NC, NACC, 1)
    return _tc_dense(acc, cnt, r, W_l)
